# Initial kernel scaffold; baseline (speedup 1.0000x reference)
import jax
import jax.numpy as jnp
from jax.experimental import pallas as pl


def kernel(x_nchw, w0, b0, w1, b1, w2, b2, w3, b3, w4, b4, w5, b5, w6, b6, w7, b7, w8, b8, w9, b9, w10, b10, w11, b11, w12, b12, w13, b13, w14, b14, w15, b15):
    raise NotImplementedError("write your pallas kernel here")


# speedup vs baseline: 1.0815x; 1.0815x over previous
from functools import partial

import jax
import jax.numpy as jnp
from jax.experimental import pallas as pl
from jax.experimental.pallas import tpu as pltpu

from reference import VggOutputs

_VMEM_BYTES = 48 * 1024 * 1024


def _row_tile(H, Wp, cin, cout, need_even):
    per_row = 6 * max(cin, 128) + 8 * max(cout, 128)
    cap = (12 << 20) // per_row
    target = max(1024, min(int(6e7) // (cin * cout), 4096))
    cand = [d for d in range(1, H + 1)
            if H % d == 0 and not (need_even and d % 2)
            and ((d * Wp) % 8 == 0 or d == H)]
    fits = [d for d in cand if d * Wp <= cap] or cand[:1]
    big = [d for d in fits if d * Wp >= target]
    return min(big) if big else max(fits)


def _conv3x3(x_flat, w, b, H, W, want_full=True, want_pool=False):
    N, _, Cin = x_flat.shape
    Cout = w.shape[-1]
    Wp = W + 2
    TR = _row_tile(H, Wp, Cin, Cout, need_even=want_pool)
    M = TR * Wp
    kcat = Cin <= 128

    GUARD = ((-Wp) % 16) + 16
    CTR = GUARD + Wp
    BOT = GUARD + (TR + 1) * Wp
    SLAB = BOT + Wp + 8

    Ho, Wo = H // 2, W // 2
    Wh = Wp // 2

    def body(xc_ref, xt_ref, xb_ref, w_ref, b_ref, *rest):
        slab = rest[-1]
        outs = rest[:-1]
        r0 = pl.program_id(1) * TR

        slab[pl.ds(0, GUARD), :] = jnp.zeros((GUARD, Cin), slab.dtype)
        slab[pl.ds(BOT + Wp, 8), :] = jnp.zeros((8, Cin), slab.dtype)
        slab[pl.ds(CTR, M), :] = xc_ref[0]

        @pl.when(r0 > 0)
        def _():
            slab[pl.ds(GUARD, Wp), :] = xt_ref[0, 0]

        @pl.when(r0 == 0)
        def _():
            slab[pl.ds(GUARD, Wp), :] = jnp.zeros((Wp, Cin), slab.dtype)

        @pl.when(r0 + TR < H)
        def _():
            slab[pl.ds(BOT, Wp), :] = xb_ref[0, 0]

        @pl.when(r0 + TR >= H)
        def _():
            slab[pl.ds(BOT, Wp), :] = jnp.zeros((Wp, Cin), slab.dtype)

        acc = jnp.zeros((M, Cout), jnp.float32)
        for ky in range(3):
            s = GUARD + ky * Wp - 1
            if kcat:
                xk = jnp.concatenate(
                    [slab[pl.ds(s, M), :], slab[pl.ds(s + 1, M), :],
                     slab[pl.ds(s + 2, M), :]], axis=1)
                acc = acc + jnp.dot(xk, w_ref[ky],
                                    preferred_element_type=jnp.float32)
            else:
                for kx in range(3):
                    acc = acc + jnp.dot(
                        slab[pl.ds(s + kx, M), :],
                        w_ref[ky, pl.ds(kx * Cin, Cin), :],
                        preferred_element_type=jnp.float32)

        yb = jnp.maximum(acc + b_ref[...], 0.0).astype(jnp.bfloat16)
        oi = 0
        if want_full:
            if want_pool:
                outs[oi][0] = yb
            else:
                col = jax.lax.broadcasted_iota(jnp.int32, (M, 1), 0) % Wp
                outs[oi][0] = jnp.where(col < W, yb, jnp.zeros_like(yb))
            oi += 1
        if want_pool:
            o = outs[oi]
            y4 = yb.reshape(TR // 2, 2, Wh, 2, Cout)
            p = jnp.maximum(jnp.maximum(y4[:, 0, :, 0], y4[:, 0, :, 1]),
                            jnp.maximum(y4[:, 1, :, 0], y4[:, 1, :, 1]))
            pcol = jax.lax.broadcasted_iota(jnp.int32, (1, Wh, 1), 1)
            p = jnp.where(pcol < Wo, p, jnp.zeros_like(p))
            o[0, :, pl.ds(0, Wh), :] = p
            o[0, :, pl.ds(Wh, 1), :] = jnp.zeros((TR // 2, 1, Cout), o.dtype)

    x4 = x_flat.reshape(N, H, Wp, Cin)
    w3 = w.reshape(3, 3 * Cin, Cout).astype(jnp.bfloat16)
    b2 = b.reshape(1, Cout).astype(jnp.float32)

    out_shapes, out_specs = [], []
    if want_full:
        out_shapes.append(jax.ShapeDtypeStruct((N, H * Wp, Cout), jnp.bfloat16))
        out_specs.append(pl.BlockSpec((1, M, Cout), lambda n, h: (n, h, 0)))
    if want_pool:
        out_shapes.append(
            jax.ShapeDtypeStruct((N, Ho, Wo + 2, Cout), jnp.bfloat16))
        out_specs.append(
            pl.BlockSpec((1, TR // 2, Wo + 2, Cout), lambda n, h: (n, h, 0, 0)))

    res = pl.pallas_call(
        body,
        out_shape=out_shapes,
        grid=(N, H // TR),
        in_specs=[
            pl.BlockSpec((1, M, Cin), lambda n, h: (n, h, 0)),
            pl.BlockSpec((1, 1, Wp, Cin),
                         lambda n, h: (n, jnp.maximum(h * TR - 1, 0), 0, 0)),
            pl.BlockSpec((1, 1, Wp, Cin),
                         lambda n, h: (n, jnp.minimum(h * TR + TR, H - 1),
                                       0, 0)),
            pl.BlockSpec((3, 3 * Cin, Cout), lambda n, h: (0, 0, 0)),
            pl.BlockSpec((1, Cout), lambda n, h: (0, 0)),
        ],
        out_specs=out_specs,
        scratch_shapes=[pltpu.VMEM((SLAB, Cin), jnp.bfloat16)],
        compiler_params=pltpu.CompilerParams(
            dimension_semantics=("parallel", "parallel"),
            vmem_limit_bytes=_VMEM_BYTES,
        ),
    )(x_flat, x4, x4, w3, b2)

    full = res[0] if want_full else None
    pooled = res[-1].reshape(N, Ho * (Wo + 2), Cout) if want_pool else None
    return full, pooled


def _first_conv(x_nhwc, w, b):
    N, H, W, _ = x_nhwc.shape
    Wp = W + 2
    Cout = w.shape[-1]
    xp = jnp.pad(x_nhwc.astype(jnp.bfloat16),
                 ((0, 0), (1, 1), (1, 1), (0, 0)))
    taps = [xp[:, ky:ky + H, kx:kx + W, :]
            for ky in range(3) for kx in range(3)]
    patches = jnp.pad(jnp.concatenate(taps, axis=-1),
                      ((0, 0), (0, 0), (0, 2), (0, 0)))
    x_flat = patches.reshape(N, H * Wp, 27)

    TR = _row_tile(H, Wp, 27, Cout, need_even=False)
    M = TR * Wp

    def body(x_ref, w_ref, b_ref, o_ref):
        acc = jnp.dot(x_ref[0], w_ref[...], preferred_element_type=jnp.float32)
        yb = jnp.maximum(acc + b_ref[...], 0.0).astype(jnp.bfloat16)
        col = jax.lax.broadcasted_iota(jnp.int32, (M, 1), 0) % Wp
        o_ref[0] = jnp.where(col < W, yb, jnp.zeros_like(yb))

    return pl.pallas_call(
        body,
        out_shape=jax.ShapeDtypeStruct((N, H * Wp, Cout), jnp.bfloat16),
        grid=(N, H // TR),
        in_specs=[
            pl.BlockSpec((1, M, 27), lambda n, h: (n, h, 0)),
            pl.BlockSpec((27, Cout), lambda n, h: (0, 0)),
            pl.BlockSpec((1, Cout), lambda n, h: (0, 0)),
        ],
        out_specs=pl.BlockSpec((1, M, Cout), lambda n, h: (n, h, 0)),
        compiler_params=pltpu.CompilerParams(
            dimension_semantics=("parallel", "parallel"),
            vmem_limit_bytes=_VMEM_BYTES,
        ),
    )(x_flat, w.reshape(27, Cout).astype(jnp.bfloat16),
      b.reshape(1, Cout).astype(jnp.float32))


def _tap(x_flat, H, W):
    N, _, C = x_flat.shape
    t = x_flat.reshape(N, H, W + 2, C)[:, :, :W, :]
    return jnp.transpose(t, (0, 3, 1, 2)).astype(jnp.float32)


@jax.jit
def kernel(x_nchw, w0, b0, w1, b1, w2, b2, w3, b3, w4, b4, w5, b5, w6, b6,
           w7, b7, w8, b8, w9, b9, w10, b10, w11, b11, w12, b12, w13, b13,
           w14, b14, w15, b15):
    x_nhwc = jnp.transpose(x_nchw, (0, 2, 3, 1))

    a = _first_conv(x_nhwc, w0, b0)
    _, a = _conv3x3(a, w1, b1, 224, 224,
                    want_full=False, want_pool=True)
    a, _ = _conv3x3(a, w2, b2, 112, 112)
    t22, a = _conv3x3(a, w3, b3, 112, 112, want_pool=True)
    a, _ = _conv3x3(a, w4, b4, 56, 56)
    a, _ = _conv3x3(a, w5, b5, 56, 56)
    t32 = a
    a, _ = _conv3x3(a, w6, b6, 56, 56)
    _, a = _conv3x3(a, w7, b7, 56, 56,
                    want_full=False, want_pool=True)
    a, _ = _conv3x3(a, w8, b8, 28, 28)
    a, _ = _conv3x3(a, w9, b9, 28, 28)
    t42 = a
    a, _ = _conv3x3(a, w10, b10, 28, 28)
    _, a = _conv3x3(a, w11, b11, 28, 28,
                    want_full=False, want_pool=True)
    a, _ = _conv3x3(a, w12, b12, 14, 14)
    a, _ = _conv3x3(a, w13, b13, 14, 14)
    a, _ = _conv3x3(a, w14, b14, 14, 14)
    a, _ = _conv3x3(a, w15, b15, 14, 14)

    return VggOutputs(_tap(t22, 112, 112), _tap(t32, 56, 56),
                      _tap(t42, 28, 28), _tap(a, 14, 14))


# speedup vs baseline: 1.1288x; 1.0437x over previous
from functools import partial

import jax
import jax.numpy as jnp
from jax.experimental import pallas as pl
from jax.experimental.pallas import tpu as pltpu

from reference import VggOutputs

_VMEM_BYTES = 48 * 1024 * 1024


def _row_tile(H, Wp, cin, cout, need_even):
    per_row = 6 * max(cin, 128) + 8 * max(cout, 128)
    cap = (12 << 20) // per_row
    target = max(1024, min(int(6e7) // (cin * cout), 4096))
    cand = [d for d in range(1, H + 1)
            if H % d == 0 and not (need_even and d % 2)
            and ((d * Wp) % 8 == 0 or d == H)]
    fits = [d for d in cand if d * Wp <= cap] or cand[:1]
    big = [d for d in fits if d * Wp >= target]
    return min(big) if big else max(fits)


def _conv3x3(x_flat, w, b, H, W, want_full=True, want_pool=False):
    N, _, Cin = x_flat.shape
    Cout = w.shape[-1]
    Wp = W + 2
    TR = _row_tile(H, Wp, Cin, Cout, need_even=want_pool)
    M = TR * Wp

    GUARD = ((-Wp) % 16) + 16
    CTR = GUARD + Wp
    BOT = GUARD + (TR + 1) * Wp
    SLAB = BOT + Wp + 8

    Ho, Wo = H // 2, W // 2
    Wh = Wp // 2

    def body(xc_ref, xt_ref, xb_ref, w_ref, b_ref, *rest):
        slab = rest[-1]
        outs = rest[:-1]
        r0 = pl.program_id(1) * TR

        slab[pl.ds(0, GUARD), :] = jnp.zeros((GUARD, Cin), slab.dtype)
        slab[pl.ds(BOT + Wp, 8), :] = jnp.zeros((8, Cin), slab.dtype)
        slab[pl.ds(CTR, M), :] = xc_ref[0]

        @pl.when(r0 > 0)
        def _():
            slab[pl.ds(GUARD, Wp), :] = xt_ref[0, 0]

        @pl.when(r0 == 0)
        def _():
            slab[pl.ds(GUARD, Wp), :] = jnp.zeros((Wp, Cin), slab.dtype)

        @pl.when(r0 + TR < H)
        def _():
            slab[pl.ds(BOT, Wp), :] = xb_ref[0, 0]

        @pl.when(r0 + TR >= H)
        def _():
            slab[pl.ds(BOT, Wp), :] = jnp.zeros((Wp, Cin), slab.dtype)

        acc = jnp.zeros((M, Cout), jnp.float32)
        for ky in range(3):
            s = GUARD + ky * Wp - 1
            for kx in range(3):
                acc = acc + jnp.dot(
                    slab[pl.ds(s + kx, M), :],
                    w_ref[ky, pl.ds(kx * Cin, Cin), :],
                    preferred_element_type=jnp.float32)

        yb = jnp.maximum(acc + b_ref[...], 0.0).astype(jnp.bfloat16)
        oi = 0
        if want_full:
            o = outs[oi]
            o[0] = yb.reshape(TR, Wp, Cout)
            if not want_pool:
                o[0, :, pl.ds(W, 2), :] = jnp.zeros((TR, 2, Cout), o.dtype)
            oi += 1
        if want_pool:
            o = outs[oi]
            y4 = yb.reshape(TR // 2, 2, Wh, 2, Cout)
            p = jnp.maximum(jnp.maximum(y4[:, 0, :, 0], y4[:, 0, :, 1]),
                            jnp.maximum(y4[:, 1, :, 0], y4[:, 1, :, 1]))
            o[0, :, pl.ds(0, Wh), :] = p
            o[0, :, pl.ds(Wo, 2), :] = jnp.zeros((TR // 2, 2, Cout), o.dtype)

    x4 = x_flat.reshape(N, H, Wp, Cin)
    w3 = w.reshape(3, 3 * Cin, Cout).astype(jnp.bfloat16)
    b2 = b.reshape(1, Cout).astype(jnp.float32)

    out_shapes, out_specs = [], []
    if want_full:
        out_shapes.append(
            jax.ShapeDtypeStruct((N, H, Wp, Cout), jnp.bfloat16))
        out_specs.append(
            pl.BlockSpec((1, TR, Wp, Cout), lambda n, h: (n, h, 0, 0)))
    if want_pool:
        out_shapes.append(
            jax.ShapeDtypeStruct((N, Ho, Wo + 2, Cout), jnp.bfloat16))
        out_specs.append(
            pl.BlockSpec((1, TR // 2, Wo + 2, Cout), lambda n, h: (n, h, 0, 0)))

    res = pl.pallas_call(
        body,
        out_shape=out_shapes,
        grid=(N, H // TR),
        in_specs=[
            pl.BlockSpec((1, M, Cin), lambda n, h: (n, h, 0)),
            pl.BlockSpec((1, 1, Wp, Cin),
                         lambda n, h: (n, jnp.maximum(h * TR - 1, 0), 0, 0)),
            pl.BlockSpec((1, 1, Wp, Cin),
                         lambda n, h: (n, jnp.minimum(h * TR + TR, H - 1),
                                       0, 0)),
            pl.BlockSpec((3, 3 * Cin, Cout), lambda n, h: (0, 0, 0)),
            pl.BlockSpec((1, Cout), lambda n, h: (0, 0)),
        ],
        out_specs=out_specs,
        scratch_shapes=[pltpu.VMEM((SLAB, Cin), jnp.bfloat16)],
        compiler_params=pltpu.CompilerParams(
            dimension_semantics=("parallel", "parallel"),
            vmem_limit_bytes=_VMEM_BYTES,
        ),
    )(x_flat, x4, x4, w3, b2)

    full = res[0].reshape(N, H * Wp, Cout) if want_full else None
    pooled = res[-1].reshape(N, Ho * (Wo + 2), Cout) if want_pool else None
    return full, pooled


def _first_conv(x_nhwc, w, b):
    N, H, W, _ = x_nhwc.shape
    Wp = W + 2
    Cout = w.shape[-1]
    xp = jnp.pad(x_nhwc.astype(jnp.bfloat16),
                 ((0, 0), (1, 1), (1, 1), (0, 0)))
    taps = [xp[:, ky:ky + H, kx:kx + W, :]
            for ky in range(3) for kx in range(3)]
    patches = jnp.pad(jnp.concatenate(taps, axis=-1),
                      ((0, 0), (0, 0), (0, 2), (0, 0)))
    x_flat = patches.reshape(N, H * Wp, 27)

    TR = _row_tile(H, Wp, 27, Cout, need_even=False)
    M = TR * Wp

    def body(x_ref, w_ref, b_ref, o_ref):
        acc = jnp.dot(x_ref[0], w_ref[...], preferred_element_type=jnp.float32)
        yb = jnp.maximum(acc + b_ref[...], 0.0).astype(jnp.bfloat16)
        o_ref[0] = yb.reshape(TR, Wp, Cout)
        o_ref[0, :, pl.ds(W, 2), :] = jnp.zeros((TR, 2, Cout), o_ref.dtype)

    out = pl.pallas_call(
        body,
        out_shape=jax.ShapeDtypeStruct((N, H, Wp, Cout), jnp.bfloat16),
        grid=(N, H // TR),
        in_specs=[
            pl.BlockSpec((1, M, 27), lambda n, h: (n, h, 0)),
            pl.BlockSpec((27, Cout), lambda n, h: (0, 0)),
            pl.BlockSpec((1, Cout), lambda n, h: (0, 0)),
        ],
        out_specs=pl.BlockSpec((1, TR, Wp, Cout), lambda n, h: (n, h, 0, 0)),
        compiler_params=pltpu.CompilerParams(
            dimension_semantics=("parallel", "parallel"),
            vmem_limit_bytes=_VMEM_BYTES,
        ),
    )(x_flat, w.reshape(27, Cout).astype(jnp.bfloat16),
      b.reshape(1, Cout).astype(jnp.float32))
    return out.reshape(N, H * Wp, Cout)


def _tap(x_flat, H, W):
    N, _, C = x_flat.shape
    t = x_flat.reshape(N, H, W + 2, C)[:, :, :W, :]
    return jnp.transpose(t, (0, 3, 1, 2)).astype(jnp.float32)


@jax.jit
def kernel(x_nchw, w0, b0, w1, b1, w2, b2, w3, b3, w4, b4, w5, b5, w6, b6,
           w7, b7, w8, b8, w9, b9, w10, b10, w11, b11, w12, b12, w13, b13,
           w14, b14, w15, b15):
    x_nhwc = jnp.transpose(x_nchw, (0, 2, 3, 1))

    a = _first_conv(x_nhwc, w0, b0)
    _, a = _conv3x3(a, w1, b1, 224, 224,
                    want_full=False, want_pool=True)
    a, _ = _conv3x3(a, w2, b2, 112, 112)
    t22, a = _conv3x3(a, w3, b3, 112, 112, want_pool=True)
    a, _ = _conv3x3(a, w4, b4, 56, 56)
    a, _ = _conv3x3(a, w5, b5, 56, 56)
    t32 = a
    a, _ = _conv3x3(a, w6, b6, 56, 56)
    _, a = _conv3x3(a, w7, b7, 56, 56,
                    want_full=False, want_pool=True)
    a, _ = _conv3x3(a, w8, b8, 28, 28)
    a, _ = _conv3x3(a, w9, b9, 28, 28)
    t42 = a
    a, _ = _conv3x3(a, w10, b10, 28, 28)
    _, a = _conv3x3(a, w11, b11, 28, 28,
                    want_full=False, want_pool=True)
    a, _ = _conv3x3(a, w12, b12, 14, 14)
    a, _ = _conv3x3(a, w13, b13, 14, 14)
    a, _ = _conv3x3(a, w14, b14, 14, 14)
    a, _ = _conv3x3(a, w15, b15, 14, 14)

    return VggOutputs(_tap(t22, 112, 112), _tap(t32, 56, 56),
                      _tap(t42, 28, 28), _tap(a, 14, 14))
